# baseline (speedup 1.0000x reference)
import functools

import jax
import jax.numpy as jnp
from jax import lax
from jax.experimental import pallas as pl
from jax.experimental.pallas import tpu as pltpu
from jax.experimental.pallas import tpu_sc as plsc

N = 10000
E = 320000
D = 128

NC = 2
NS = 16
NW = NC * NS

KB = 128
CH = 4
NBC = 20
EPT = CH * NBC * KB
EP = NW * EPT
NP = 10240
RPS = NP // NS

_mesh = plsc.VectorSubcoreMesh(core_axis_name="c", subcore_axis_name="s")


@functools.partial(
    pl.kernel,
    mesh=_mesh,
    out_type=jax.ShapeDtypeStruct((NC, NP, D), jnp.float32),
    scratch_types=[
        pltpu.VMEM((NBC, KB), jnp.int32),
        pltpu.VMEM((NBC, KB), jnp.int32),
        pltpu.VMEM((NBC, KB), jnp.float32),
        pltpu.VMEM((KB, D), jnp.float32),
        pltpu.VMEM((KB, D), jnp.float32),
        pltpu.VMEM_SHARED((NP, D), jnp.float32),
        pltpu.SemaphoreType.DMA,
        pltpu.SemaphoreType.DMA,
    ],
)
def _sc_agg(x_hbm, src_hbm, dst_hbm, adj_hbm, out_hbm,
            srcb, dstb, adjb, b0, b1, acc_sh, semg0, semg1):
    c = lax.axis_index("c")
    s = lax.axis_index("s")
    wid = s * NC + c

    zv = jnp.zeros((16,), jnp.float32)

    def zrow(i, carry):
        for k in range(D // 16):
            b0[i, pl.ds(k * 16, 16)] = zv
        return carry

    lax.fori_loop(0, KB, zrow, 0)
    for q in range(RPS // KB):
        pltpu.sync_copy(b0, acc_sh.at[pl.ds(s * RPS + q * KB, KB)])
    plsc.subcore_barrier()

    def start_gather(t, rows, sem):
        pltpu.async_copy(x_hbm.at[srcb.at[t]], rows, sem)

    def wait_gather(rows, sem):
        pltpu.make_async_copy(x_hbm.at[srcb.at[0]], rows, sem).wait()

    def scale(t, rows):
        def sbody(g, inner):
            av = adjb[t, pl.ds(g * 16, 16)]
            for e in range(16):
                a = av[e]
                j = g * 16 + e
                for k in range(D // 16):
                    sl = pl.ds(k * 16, 16)
                    rows[j, sl] = rows[j, sl] * a
            return inner

        lax.fori_loop(0, KB // 16, sbody, 0)

    def start_scatter(t, rows, sem):
        pltpu.async_copy(rows, acc_sh.at[dstb.at[t]], sem, add=True)

    def wait_scatter(rows, sem):
        pltpu.make_async_copy(rows, acc_sh.at[dstb.at[0]], sem).wait()

    def chunk_body(ch, carry):
        pltpu.sync_copy(src_hbm.at[wid, ch], srcb)
        pltpu.sync_copy(dst_hbm.at[wid, ch], dstb)
        pltpu.sync_copy(adj_hbm.at[wid, ch], adjb)
        start_gather(0, b0, semg0)

        def body(i, c2):
            t0 = 2 * i
            wait_gather(b0, semg0)
            start_gather(t0 + 1, b1, semg1)
            scale(t0, b0)
            pltpu.sync_copy(b0, acc_sh.at[dstb.at[t0]], add=True)
            wait_gather(b1, semg1)
            start_gather(t0 + 2, b0, semg0)
            scale(t0 + 1, b1)
            pltpu.sync_copy(b1, acc_sh.at[dstb.at[t0 + 1]], add=True)
            return c2

        lax.fori_loop(0, NBC // 2 - 1, body, 0)
        wait_gather(b0, semg0)
        start_gather(NBC - 1, b1, semg1)
        scale(NBC - 2, b0)
        pltpu.sync_copy(b0, acc_sh.at[dstb.at[NBC - 2]], add=True)
        wait_gather(b1, semg1)
        scale(NBC - 1, b1)
        pltpu.sync_copy(b1, acc_sh.at[dstb.at[NBC - 1]], add=True)
        return carry

    lax.fori_loop(0, CH, chunk_body, 0)
    plsc.subcore_barrier()

    for q in range(RPS // KB):
        r0 = s * RPS + q * KB
        pltpu.sync_copy(acc_sh.at[pl.ds(r0, KB)], b0)
        pltpu.sync_copy(b0, out_hbm.at[c, pl.ds(r0, KB)])


RB = 2000


def _mm_body(p_ref, w_ref, b_ref, o_ref):
    acc = p_ref[0] + p_ref[1]
    o_ref[...] = (
        jnp.dot(acc, w_ref[...], preferred_element_type=jnp.float32)
        + b_ref[...]
    )


_mm = pl.pallas_call(
    _mm_body,
    grid=(N // RB,),
    in_specs=[
        pl.BlockSpec((NC, RB, D), lambda i: (0, i, 0)),
        pl.BlockSpec((D, D), lambda i: (0, 0)),
        pl.BlockSpec((1, D), lambda i: (0, 0)),
    ],
    out_specs=pl.BlockSpec((RB, D), lambda i: (i, 0)),
    out_shape=jax.ShapeDtypeStruct((N, D), jnp.float32),
)


def kernel(x, edge_index, adj_values, W, b):
    ei = edge_index.astype(jnp.int32)
    pad = EP - E
    dst = jnp.pad(ei[0], (0, pad)).reshape(NW, CH, NBC, KB)
    src = jnp.pad(ei[1], (0, pad)).reshape(NW, CH, NBC, KB)
    adj = jnp.pad(adj_values, (0, pad)).reshape(NW, CH, NBC, KB)
    partial = _sc_agg(x, src, dst, adj)
    return _mm(partial, W.T, b.reshape(1, D))


# speedup vs baseline: 2.9027x; 2.9027x over previous
import functools

import jax
import jax.numpy as jnp
from jax import lax
from jax.experimental import pallas as pl
from jax.experimental.pallas import tpu as pltpu
from jax.experimental.pallas import tpu_sc as plsc

N = 10000
E = 320000
D = 128

NC = 2
NS = 16
NW = NC * NS

KB = 80
CH = 5
NBC = 25
EPT = CH * NBC * KB
EP = NW * EPT
NP = 10240
RPS = NP // NS

_mesh = plsc.VectorSubcoreMesh(core_axis_name="c", subcore_axis_name="s")


@functools.partial(
    pl.kernel,
    mesh=_mesh,
    out_type=jax.ShapeDtypeStruct((NC, NP, D), jnp.float32),
    scratch_types=[
        pltpu.VMEM((NBC, KB), jnp.int32),
        pltpu.VMEM((NBC, KB), jnp.int32),
        pltpu.VMEM((NBC, KB), jnp.float32),
        pltpu.VMEM((NBC, KB), jnp.int32),
        pltpu.VMEM((NBC, KB), jnp.int32),
        pltpu.VMEM((NBC, KB), jnp.float32),
        pltpu.VMEM((KB, D), jnp.float32),
        pltpu.VMEM((KB, D), jnp.float32),
        pltpu.VMEM_SHARED((NP, D), jnp.float32),
        pltpu.SemaphoreType.DMA,
        pltpu.SemaphoreType.DMA,
        pltpu.SemaphoreType.DMA,
        pltpu.SemaphoreType.DMA,
    ],
)
def _sc_agg(x_hbm, src_hbm, dst_hbm, adj_hbm, out_hbm,
            srcbA, dstbA, adjbA, srcbB, dstbB, adjbB,
            b0, b1, acc_sh, semg0, semg1, semiA, semiB):
    c = lax.axis_index("c")
    s = lax.axis_index("s")
    wid = s * NC + c

    zv = jnp.zeros((16,), jnp.float32)

    def zrow(i, carry):
        for k in range(D // 16):
            b0[i, pl.ds(k * 16, 16)] = zv
        return carry

    lax.fori_loop(0, KB, zrow, 0)
    for q in range(RPS // KB):
        pltpu.sync_copy(b0, acc_sh.at[pl.ds(s * RPS + q * KB, KB)])
    plsc.subcore_barrier()

    def start_gather(t, rows, sem):
        pltpu.async_copy(x_hbm.at[srcb.at[t]], rows, sem)

    def wait_gather(rows, sem):
        pltpu.make_async_copy(x_hbm.at[srcb.at[0]], rows, sem).wait()

    def scale(t, rows):
        def sbody(g, inner):
            av = adjb[t, pl.ds(g * 16, 16)]
            for e in range(16):
                a = av[e]
                j = g * 16 + e
                for k in range(D // 16):
                    sl = pl.ds(k * 16, 16)
                    rows[j, sl] = rows[j, sl] * a
            return inner

        lax.fori_loop(0, KB // 16, sbody, 0)

    def start_scatter(t, rows, sem):
        pltpu.async_copy(rows, acc_sh.at[dstb.at[t]], sem, add=True)

    def wait_scatter(rows, sem):
        pltpu.make_async_copy(rows, acc_sh.at[dstb.at[0]], sem).wait()

    sets = [(srcbA, dstbA, adjbA, semiA), (srcbB, dstbB, adjbB, semiB)]

    def load_idx(ch, st):
        sb, db, ab, sem = st
        pltpu.async_copy(src_hbm.at[wid, ch], sb, sem)
        pltpu.async_copy(dst_hbm.at[wid, ch], db, sem)
        pltpu.async_copy(adj_hbm.at[wid, ch], ab, sem)

    def wait_idx(st):
        sb, db, ab, sem = st
        pltpu.make_async_copy(src_hbm.at[wid, 0], sb, sem).wait()
        pltpu.make_async_copy(dst_hbm.at[wid, 0], db, sem).wait()
        pltpu.make_async_copy(adj_hbm.at[wid, 0], ab, sem).wait()

    load_idx(0, sets[0])
    for ch in range(CH):
        srcb, dstb, adjb, _ = sets[ch % 2]
        wait_idx(sets[ch % 2])
        if ch + 1 < CH:
            load_idx(ch + 1, sets[(ch + 1) % 2])

        def start_gather_c(t, rows, sem, srcb=srcb):
            pltpu.async_copy(x_hbm.at[srcb.at[t]], rows, sem)

        def wait_gather_c(rows, sem, srcb=srcb):
            pltpu.make_async_copy(x_hbm.at[srcb.at[0]], rows, sem).wait()

        def scale_c(t, rows, adjb=adjb):
            def sbody(g, inner):
                av = adjb[t, pl.ds(g * 16, 16)]
                for e in range(16):
                    a = av[e]
                    j = g * 16 + e
                    for k in range(D // 16):
                        sl = pl.ds(k * 16, 16)
                        rows[j, sl] = rows[j, sl] * a
                return inner

            lax.fori_loop(0, KB // 16, sbody, 0)

        start_gather_c(0, b0, semg0)

        def body(i, c2, dstb=dstb):
            t0 = 2 * i
            wait_gather_c(b0, semg0)
            start_gather_c(t0 + 1, b1, semg1)
            scale_c(t0, b0)
            pltpu.sync_copy(b0, acc_sh.at[dstb.at[t0]], add=True)
            wait_gather_c(b1, semg1)
            start_gather_c(t0 + 2, b0, semg0)
            scale_c(t0 + 1, b1)
            pltpu.sync_copy(b1, acc_sh.at[dstb.at[t0 + 1]], add=True)
            return c2

        lax.fori_loop(0, (NBC - 1) // 2, body, 0)
        wait_gather_c(b0, semg0)
        scale_c(NBC - 1, b0)
        pltpu.sync_copy(b0, acc_sh.at[dstb.at[NBC - 1]], add=True)

    plsc.subcore_barrier()

    for q in range(RPS // KB):
        r0 = s * RPS + q * KB
        pltpu.sync_copy(acc_sh.at[pl.ds(r0, KB)], b0)
        pltpu.sync_copy(b0, out_hbm.at[c, pl.ds(r0, KB)])


RB = 2000


def _mm_body(p_ref, w_ref, b_ref, o_ref):
    acc = p_ref[0] + p_ref[1]
    o_ref[...] = (
        jnp.dot(acc, w_ref[...], preferred_element_type=jnp.float32)
        + b_ref[...]
    )


_mm = pl.pallas_call(
    _mm_body,
    grid=(N // RB,),
    in_specs=[
        pl.BlockSpec((NC, RB, D), lambda i: (0, i, 0)),
        pl.BlockSpec((D, D), lambda i: (0, 0)),
        pl.BlockSpec((1, D), lambda i: (0, 0)),
    ],
    out_specs=pl.BlockSpec((RB, D), lambda i: (i, 0)),
    out_shape=jax.ShapeDtypeStruct((N, D), jnp.float32),
)


def kernel(x, edge_index, adj_values, W, b):
    ei = edge_index.astype(jnp.int32)
    dst = ei[0].reshape(NW, CH, NBC, KB)
    src = ei[1].reshape(NW, CH, NBC, KB)
    adj = adj_values.reshape(NW, CH, NBC, KB)
    partial = _sc_agg(x, src, dst, adj)
    return _mm(partial, W.T, b.reshape(1, D))


# speedup vs baseline: 2.9155x; 1.0044x over previous
import functools

import jax
import jax.numpy as jnp
from jax import lax
from jax.experimental import pallas as pl
from jax.experimental.pallas import tpu as pltpu
from jax.experimental.pallas import tpu_sc as plsc

N = 10000
E = 320000
D = 128

NC = 2
NS = 16
NW = NC * NS

KB = 80
CH = 5
NBC = 25
EPT = CH * NBC * KB
EP = NW * EPT
NP = 10240
RPS = NP // NS

_mesh = plsc.VectorSubcoreMesh(core_axis_name="c", subcore_axis_name="s")


@functools.partial(
    pl.kernel,
    mesh=_mesh,
    out_type=jax.ShapeDtypeStruct((NC, NP, D), jnp.float32),
    scratch_types=[
        pltpu.VMEM((NBC, KB), jnp.int32),
        pltpu.VMEM((NBC, KB), jnp.int32),
        pltpu.VMEM((NBC, KB), jnp.float32),
        pltpu.VMEM((NBC, KB), jnp.int32),
        pltpu.VMEM((NBC, KB), jnp.int32),
        pltpu.VMEM((NBC, KB), jnp.float32),
        pltpu.VMEM((KB, D), jnp.float32),
        pltpu.VMEM((KB, D), jnp.float32),
        pltpu.VMEM_SHARED((NP, D), jnp.float32),
        pltpu.SemaphoreType.DMA,
        pltpu.SemaphoreType.DMA,
        pltpu.SemaphoreType.DMA,
        pltpu.SemaphoreType.DMA,
    ],
)
def _sc_agg(x_hbm, src_hbm, dst_hbm, adj_hbm, out_hbm,
            srcbA, dstbA, adjbA, srcbB, dstbB, adjbB,
            b0, b1, acc_sh, semg0, semg1, semiA, semiB):
    c = lax.axis_index("c")
    s = lax.axis_index("s")
    wid = s * NC + c

    zv = jnp.zeros((16,), jnp.float32)

    def zrow(i, carry):
        for k in range(D // 16):
            b0[i, pl.ds(k * 16, 16)] = zv
        return carry

    lax.fori_loop(0, KB, zrow, 0)
    for q in range(RPS // KB):
        pltpu.sync_copy(b0, acc_sh.at[pl.ds(s * RPS + q * KB, KB)])
    plsc.subcore_barrier()

    def start_gather(t, rows, sem):
        pltpu.async_copy(x_hbm.at[srcb.at[t]], rows, sem)

    def wait_gather(rows, sem):
        pltpu.make_async_copy(x_hbm.at[srcb.at[0]], rows, sem).wait()

    def scale(t, rows):
        def sbody(g, inner):
            av = adjb[t, pl.ds(g * 16, 16)]
            for e in range(16):
                a = av[e]
                j = g * 16 + e
                for k in range(D // 16):
                    sl = pl.ds(k * 16, 16)
                    rows[j, sl] = rows[j, sl] * a
            return inner

        lax.fori_loop(0, KB // 16, sbody, 0)

    def start_scatter(t, rows, sem):
        pltpu.async_copy(rows, acc_sh.at[dstb.at[t]], sem, add=True)

    def wait_scatter(rows, sem):
        pltpu.make_async_copy(rows, acc_sh.at[dstb.at[0]], sem).wait()

    sets = [(srcbA, dstbA, adjbA, semiA), (srcbB, dstbB, adjbB, semiB)]

    def load_idx(ch, st):
        sb, db, ab, sem = st
        pltpu.async_copy(src_hbm.at[wid, ch], sb, sem)
        pltpu.async_copy(dst_hbm.at[wid, ch], db, sem)
        pltpu.async_copy(adj_hbm.at[wid, ch], ab, sem)

    def wait_idx(st):
        sb, db, ab, sem = st
        pltpu.make_async_copy(src_hbm.at[wid, 0], sb, sem).wait()
        pltpu.make_async_copy(dst_hbm.at[wid, 0], db, sem).wait()
        pltpu.make_async_copy(adj_hbm.at[wid, 0], ab, sem).wait()

    load_idx(0, sets[0])
    for ch in range(CH):
        srcb, dstb, adjb, _ = sets[ch % 2]
        wait_idx(sets[ch % 2])
        if ch + 1 < CH:
            load_idx(ch + 1, sets[(ch + 1) % 2])

        def start_gather_c(t, rows, sem, srcb=srcb):
            pltpu.async_copy(x_hbm.at[srcb.at[t]], rows, sem)

        def wait_gather_c(rows, sem, srcb=srcb):
            pltpu.make_async_copy(x_hbm.at[srcb.at[0]], rows, sem).wait()

        def scale_c(t, rows, adjb=adjb):
            def sbody(g, inner):
                av = adjb[t, pl.ds(g * 16, 16)]
                for e in range(16):
                    a = av[e]
                    j = g * 16 + e
                    for k in range(D // 16):
                        sl = pl.ds(k * 16, 16)
                        rows[j, sl] = rows[j, sl] * a
                return inner

            lax.fori_loop(0, KB // 16, sbody, 0)

        start_gather_c(0, b0, semg0)

        def body(i, c2, dstb=dstb):
            t0 = 2 * i
            wait_gather_c(b0, semg0)
            start_gather_c(t0 + 1, b1, semg1)
            scale_c(t0, b0)
            pltpu.sync_copy(b0, acc_sh.at[dstb.at[t0]], add=True)
            wait_gather_c(b1, semg1)
            start_gather_c(t0 + 2, b0, semg0)
            scale_c(t0 + 1, b1)
            pltpu.sync_copy(b1, acc_sh.at[dstb.at[t0 + 1]], add=True)
            return c2

        lax.fori_loop(0, (NBC - 1) // 2, body, 0)
        wait_gather_c(b0, semg0)
        scale_c(NBC - 1, b0)
        pltpu.sync_copy(b0, acc_sh.at[dstb.at[NBC - 1]], add=True)

    plsc.subcore_barrier()

    pltpu.sync_copy(acc_sh.at[pl.ds(s * RPS, RPS)],
                    out_hbm.at[c, pl.ds(s * RPS, RPS)])


RB = 2000


def _mm_body(p_ref, w_ref, b_ref, o_ref):
    acc = p_ref[0] + p_ref[1]
    o_ref[...] = (
        jnp.dot(acc, w_ref[...], preferred_element_type=jnp.float32)
        + b_ref[...]
    )


_mm = pl.pallas_call(
    _mm_body,
    grid=(N // RB,),
    in_specs=[
        pl.BlockSpec((NC, RB, D), lambda i: (0, i, 0)),
        pl.BlockSpec((D, D), lambda i: (0, 0)),
        pl.BlockSpec((1, D), lambda i: (0, 0)),
    ],
    out_specs=pl.BlockSpec((RB, D), lambda i: (i, 0)),
    out_shape=jax.ShapeDtypeStruct((N, D), jnp.float32),
)


def kernel(x, edge_index, adj_values, W, b):
    ei = edge_index.astype(jnp.int32)
    dst = ei[0].reshape(NW, CH, NBC, KB)
    src = ei[1].reshape(NW, CH, NBC, KB)
    adj = adj_values.reshape(NW, CH, NBC, KB)
    partial = _sc_agg(x, src, dst, adj)
    return _mm(partial, W.T, b.reshape(1, D))


# speedup vs baseline: 2.9353x; 1.0068x over previous
import functools

import jax
import jax.numpy as jnp
from jax import lax
from jax.experimental import pallas as pl
from jax.experimental.pallas import tpu as pltpu
from jax.experimental.pallas import tpu_sc as plsc

N = 10000
E = 320000
D = 128

NC = 2
NS = 16
NW = NC * NS

KB = 80
CH = 5
NBC = 25
EPT = CH * NBC * KB
EP = NW * EPT
NP = 10240
RPS = NP // NS

_mesh = plsc.VectorSubcoreMesh(core_axis_name="c", subcore_axis_name="s")


@functools.partial(
    pl.kernel,
    mesh=_mesh,
    out_type=jax.ShapeDtypeStruct((NC, NP, D), jnp.float32),
    scratch_types=[
        pltpu.VMEM((NBC, KB), jnp.int32),
        pltpu.VMEM((NBC, KB), jnp.int32),
        pltpu.VMEM((NBC, KB), jnp.float32),
        pltpu.VMEM((NBC, KB), jnp.int32),
        pltpu.VMEM((NBC, KB), jnp.int32),
        pltpu.VMEM((NBC, KB), jnp.float32),
        pltpu.VMEM((KB, D), jnp.float32),
        pltpu.VMEM((KB, D), jnp.float32),
        pltpu.VMEM_SHARED((NP, D), jnp.float32),
        pltpu.SemaphoreType.DMA,
        pltpu.SemaphoreType.DMA,
        pltpu.SemaphoreType.DMA,
        pltpu.SemaphoreType.DMA,
    ],
)
def _sc_agg(x_hbm, src_hbm, dst_hbm, adj_hbm, out_hbm,
            srcbA, dstbA, adjbA, srcbB, dstbB, adjbB,
            b0, b1, acc_sh, semg0, semg1, semiA, semiB):
    c = lax.axis_index("c")
    s = lax.axis_index("s")
    wid = s * NC + c

    zv = jnp.zeros((16,), jnp.float32)

    def zrow(i, carry):
        for k in range(D // 16):
            b0[i, pl.ds(k * 16, 16)] = zv
        return carry

    lax.fori_loop(0, KB, zrow, 0)
    for q in range(RPS // KB):
        pltpu.sync_copy(b0, acc_sh.at[pl.ds(s * RPS + q * KB, KB)])
    plsc.subcore_barrier()

    def start_gather(t, rows, sem):
        pltpu.async_copy(x_hbm.at[srcb.at[t]], rows, sem)

    def wait_gather(rows, sem):
        pltpu.make_async_copy(x_hbm.at[srcb.at[0]], rows, sem).wait()

    def scale(t, rows):
        def sbody(g, inner):
            av = adjb[t, pl.ds(g * 16, 16)]
            for e in range(16):
                a = av[e]
                j = g * 16 + e
                for k in range(D // 16):
                    sl = pl.ds(k * 16, 16)
                    rows[j, sl] = rows[j, sl] * a
            return inner

        lax.fori_loop(0, KB // 16, sbody, 0)

    def start_scatter(t, rows, sem):
        pltpu.async_copy(rows, acc_sh.at[dstb.at[t]], sem, add=True)

    def wait_scatter(rows, sem):
        pltpu.make_async_copy(rows, acc_sh.at[dstb.at[0]], sem).wait()

    sets = [(srcbA, dstbA, adjbA, semiA), (srcbB, dstbB, adjbB, semiB)]

    def load_idx(ch, st):
        sb, db, ab, sem = st
        pltpu.async_copy(src_hbm.at[wid, ch], sb, sem)
        pltpu.async_copy(dst_hbm.at[wid, ch], db, sem)
        pltpu.async_copy(adj_hbm.at[wid, ch], ab, sem)

    def wait_idx(st):
        sb, db, ab, sem = st
        pltpu.make_async_copy(src_hbm.at[wid, 0], sb, sem).wait()
        pltpu.make_async_copy(dst_hbm.at[wid, 0], db, sem).wait()
        pltpu.make_async_copy(adj_hbm.at[wid, 0], ab, sem).wait()

    load_idx(0, sets[0])
    for ch in range(CH):
        srcb, dstb, adjb, _ = sets[ch % 2]
        wait_idx(sets[ch % 2])
        if ch + 1 < CH:
            load_idx(ch + 1, sets[(ch + 1) % 2])

        def start_gather_c(t, rows, sem, srcb=srcb):
            pltpu.async_copy(x_hbm.at[srcb.at[t]], rows, sem)

        def wait_gather_c(rows, sem, srcb=srcb):
            pltpu.make_async_copy(x_hbm.at[srcb.at[0]], rows, sem).wait()

        def scale_c(t, rows, adjb=adjb):
            def sbody(g, inner):
                av = adjb[t, pl.ds(g * 16, 16)]
                for e in range(16):
                    a = av[e]
                    j = g * 16 + e
                    for k in range(D // 16):
                        sl = pl.ds(k * 16, 16)
                        rows[j, sl] = rows[j, sl] * a
                return inner

            lax.fori_loop(0, KB // 16, sbody, 0)

        start_gather_c(0, b0, semg0)

        def body(i, c2, dstb=dstb):
            t0 = 2 * i
            wait_gather_c(b0, semg0)
            start_gather_c(t0 + 1, b1, semg1)
            scale_c(t0, b0)
            pltpu.sync_copy(b0, acc_sh.at[dstb.at[t0]], add=True)
            wait_gather_c(b1, semg1)
            start_gather_c(t0 + 2, b0, semg0)
            scale_c(t0 + 1, b1)
            pltpu.sync_copy(b1, acc_sh.at[dstb.at[t0 + 1]], add=True)
            return c2

        lax.fori_loop(0, (NBC - 1) // 2, body, 0)
        wait_gather_c(b0, semg0)
        scale_c(NBC - 1, b0)
        pltpu.sync_copy(b0, acc_sh.at[dstb.at[NBC - 1]], add=True)

    plsc.subcore_barrier()

    pltpu.sync_copy(acc_sh.at[pl.ds(s * RPS, RPS)],
                    out_hbm.at[c, pl.ds(s * RPS, RPS)])


RB = 10000


def _mm_body(p_ref, w_ref, b_ref, o_ref):
    acc = p_ref[0] + p_ref[1]
    o_ref[...] = (
        jnp.dot(acc, w_ref[...], preferred_element_type=jnp.float32)
        + b_ref[...]
    )


_mm = pl.pallas_call(
    _mm_body,
    grid=(N // RB,),
    in_specs=[
        pl.BlockSpec((NC, RB, D), lambda i: (0, i, 0)),
        pl.BlockSpec((D, D), lambda i: (0, 0)),
        pl.BlockSpec((1, D), lambda i: (0, 0)),
    ],
    out_specs=pl.BlockSpec((RB, D), lambda i: (i, 0)),
    out_shape=jax.ShapeDtypeStruct((N, D), jnp.float32),
)


def kernel(x, edge_index, adj_values, W, b):
    ei = edge_index.astype(jnp.int32)
    dst = ei[0].reshape(NW, CH, NBC, KB)
    src = ei[1].reshape(NW, CH, NBC, KB)
    adj = adj_values.reshape(NW, CH, NBC, KB)
    partial = _sc_agg(x, src, dst, adj)
    return _mm(partial, W.T, b.reshape(1, D))
